# baseline (speedup 1.0000x reference)
import functools

import jax
import jax.numpy as jnp
import numpy as np
import ml_dtypes
from jax import lax
from jax.experimental import pallas as pl
from jax.experimental.pallas import tpu as pltpu
from jax.experimental.pallas import tpu_sc as plsc


_L = 3000
_PATCH = 75
_NP_ = _L // _PATCH
_KC = _L // 2 + 1
_KPAD = 1536
_DPAD = 48
_TW = 128
_VROWS = 1504
_NC, _NS = 2, 16
_NW = _NC * _NS


def _patch_hits_table(L, patch_size, num_patches):
    freqs = np.fft.fftfreq(L)
    with np.errstate(divide="ignore"):
        periods = np.abs(1.0 / freqs)
    periods[np.isinf(periods)] = 0
    hits = np.zeros((L, num_patches), dtype=bool)
    for k in range(L):
        p = periods[k]
        if p == 0:
            continue
        interval = int(p)
        peaks = np.arange(0, L, interval)
        pidx = np.floor(peaks / patch_size).astype(np.int64)
        pidx = np.unique(pidx[pidx < num_patches])
        hits[k, pidx] = True
    return hits


@functools.lru_cache(maxsize=1)
def _constants():
    t = np.arange(_L, dtype=np.int64)
    k = np.arange(_KPAD, dtype=np.int64)
    ang = (2.0 * np.pi / _L) * ((t[:, None] * k[None, :]) % _L).astype(np.float64)
    d = np.concatenate([np.cos(ang), np.sin(ang)], axis=1).astype(np.float32)
    d_hi = d.astype(ml_dtypes.bfloat16)
    d_lo = (d - d_hi.astype(np.float32)).astype(ml_dtypes.bfloat16)
    hits = _patch_hits_table(_L, _PATCH, _NP_)[:_KC]
    table = np.zeros((_VROWS, _TW), dtype=np.float32)
    table[:_KC, :_NP_] = hits.astype(np.float32)
    return d_hi, d_lo, table


def _tc_body(x_ref, dh_ref, dl_ref, sel_ref):
    xb = x_ref[0]
    xh = xb.astype(jnp.bfloat16)
    xl = (xb - xh.astype(jnp.float32)).astype(jnp.bfloat16)
    z = (jnp.dot(xh, dh_ref[...], preferred_element_type=jnp.float32)
         + jnp.dot(xh, dl_ref[...], preferred_element_type=jnp.float32)
         + jnp.dot(xl, dh_ref[...], preferred_element_type=jnp.float32))
    zc = z[:, :_KPAD]
    zs = z[:, _KPAD:]
    e = zc * zc + zs * zs
    kiota = jax.lax.broadcasted_iota(jnp.int32, (128, _KPAD), 1)
    e = jnp.where(kiota < _KC, e, -1.0)

    idxs, mults = [], []
    for _ in range(4):
        mx = jnp.max(e, axis=1, keepdims=True)
        cand = jnp.where(e == mx, kiota, jnp.int32(1 << 20))
        ij = jnp.min(cand, axis=1, keepdims=True)
        e = jnp.where(kiota == ij, -1.0, e)
        idxs.append(ij)
        mults.append(jnp.where((ij == 0) | (ij == _L // 2), 1, 2).astype(jnp.int32))

    c0 = jnp.zeros_like(mults[0])
    c1 = mults[0]
    c2 = c1 + mults[1]
    c3 = c2 + mults[2]
    starts = [c0, c1, c2, c3]

    picks = []
    for p in (1, 3, 5):
        picks.append(sum(
            idxs[j] * ((starts[j] <= p) & (p < starts[j] + mults[j])).astype(jnp.int32)
            for j in range(4)
        ))

    jiota = jax.lax.broadcasted_iota(jnp.int32, (128, 4), 1)
    selblk = jnp.where(jiota == 0, picks[0],
                       jnp.where(jiota == 1, picks[1],
                                 jnp.where(jiota == 2, picks[2], 0)))
    sel_ref[0] = selblk


def _tc_select(x, d_hi, d_lo):
    B, N, L = x.shape
    return pl.pallas_call(
        _tc_body,
        grid=(B,),
        in_specs=[
            pl.BlockSpec((1, N, L), lambda b: (b, 0, 0)),
            pl.BlockSpec((L, 2 * _KPAD), lambda b: (0, 0)),
            pl.BlockSpec((L, 2 * _KPAD), lambda b: (0, 0)),
        ],
        out_specs=pl.BlockSpec((1, N, 4), lambda b: (b, 0, 0)),
        out_shape=jax.ShapeDtypeStruct((B, N, 4), jnp.int32),
    )(x, d_hi, d_lo)


def _sc_accum_body(sel_hbm, table_hbm, wvec_hbm, out_hbm, idx_v, rows_v, acc_v, wv_v, sem):
    wid = lax.axis_index("s") * _NC + lax.axis_index("c")
    pltpu.sync_copy(sel_hbm.at[wid], idx_v)
    cps = [
        pltpu.async_copy(table_hbm.at[idx_v.at[j]], rows_v.at[pl.ds(j * 128, 128)], sem)
        for j in range(4)
    ]
    for cp in cps:
        cp.wait()
    pltpu.sync_copy(wvec_hbm, wv_v)

    def body(r, carry):
        return tuple(carry[c] + rows_v[r, pl.ds(16 * c, 16)] for c in range(3))

    z16 = jnp.zeros((16,), jnp.float32)
    acc = lax.fori_loop(0, 512, body, (z16, z16, z16))
    w16 = wv_v[...]
    for c in range(3):
        acc_v[pl.ds(16 * c, 16)] = acc[c] * w16
    pltpu.sync_copy(acc_v, out_hbm.at[wid])


@functools.lru_cache(maxsize=1)
def _sc_accum():
    mesh = plsc.VectorSubcoreMesh(
        core_axis_name="c", subcore_axis_name="s", num_cores=_NC, num_subcores=_NS
    )
    return pl.kernel(
        _sc_accum_body,
        out_type=jax.ShapeDtypeStruct((_NW, _DPAD), jnp.float32),
        mesh=mesh,
        scratch_types=[
            pltpu.VMEM((4, 128), jnp.int32),
            pltpu.VMEM((512, _TW), jnp.float32),
            pltpu.VMEM((_DPAD,), jnp.float32),
            pltpu.VMEM((16,), jnp.float32),
            pltpu.SemaphoreType.DMA,
        ],
    )


def kernel(x, weights):
    B, N, L = x.shape
    d_hi, d_lo, table = _constants()
    sel = _tc_select(x, jnp.asarray(d_hi), jnp.asarray(d_lo))
    sel2 = sel.reshape(B, 4, 128)
    wvec = jnp.full((16,), weights, jnp.float32)
    out48 = _sc_accum()(sel2, jnp.asarray(table), wvec)
    return out48[:, :_NP_]
